# baseline (speedup 1.0000x reference)
import functools

import jax
import jax.numpy as jnp
from jax.experimental import pallas as pl
from jax.experimental.pallas import tpu as pltpu

_K = 10
_N = 8192
_D = 256
_BR = 256
_NBLK = _N // _BR
_RG = 8
_NG = _BR // _RG
_NLANE = 128
_NCOL = _N // _NLANE


def _knn_kernel(x_ref, idx_ref, val_ref, xn_ref, s_ref):
    i = pl.program_id(0)

    @pl.when(i == 0)
    def _normalize():
        x = x_ref[...]
        n2 = jnp.sum(x * x, axis=1, keepdims=True)
        xn_ref[...] = x / jnp.sqrt(n2)

    xn_blk = xn_ref[pl.ds(i * _BR, _BR), :]
    s_ref[...] = jax.lax.dot_general(
        xn_blk, xn_ref[...], (((1,), (1,)), ((), ())),
        preferred_element_type=jnp.float32)

    lane = jax.lax.broadcasted_iota(jnp.int32, (_RG, _NLANE), 1)
    neg = jnp.full((_RG, _NLANE), -jnp.inf, jnp.float32)
    zero = jnp.zeros((_RG, _NLANE), jnp.int32)

    def row_group(r, _):
        base = pl.multiple_of(r * _RG, _RG)
        vals = [neg] * _K
        inds = [zero] * _K
        for g in range(_NCOL):
            v = s_ref[pl.ds(base, _RG), g * _NLANE:(g + 1) * _NLANE]
            ind = g * _NLANE + lane
            for j in range(_K):
                c = v > vals[j]
                nv = jnp.where(c, v, vals[j])
                ni = jnp.where(c, ind, inds[j])
                v = jnp.where(c, vals[j], v)
                ind = jnp.where(c, inds[j], ind)
                vals[j] = nv
                inds[j] = ni

        cv = jnp.concatenate(vals, axis=1)
        ci = jnp.concatenate(inds, axis=1)
        picked = []
        for _ in range(_K):
            m = jnp.max(cv, axis=1, keepdims=True)
            hit = cv == m
            idx = jnp.min(jnp.where(hit, ci, _N), axis=1, keepdims=True)
            picked.append(idx)
            cv = jnp.where(hit & (ci == idx), -jnp.inf, cv)
        idx_ref[pl.ds(base, _RG), :] = jnp.concatenate(picked, axis=1)
        return 0

    jax.lax.fori_loop(0, _NG, row_group, 0, unroll=False)

    row_sum = jnp.float32(1e-7) + jnp.float32(_K)
    r_inv_sqrt = row_sum ** -0.5
    val_ref[...] = jnp.full((_BR, _K), r_inv_sqrt * r_inv_sqrt, jnp.float32)


@functools.partial(jax.jit)
def kernel(mm_embedding):
    knn_ind, vals = pl.pallas_call(
        _knn_kernel,
        grid=(_NBLK,),
        in_specs=[pl.BlockSpec((_N, _D), lambda i: (0, 0))],
        out_specs=[
            pl.BlockSpec((_BR, _K), lambda i: (i, 0)),
            pl.BlockSpec((_BR, _K), lambda i: (i, 0)),
        ],
        out_shape=[
            jax.ShapeDtypeStruct((_N, _K), jnp.int32),
            jax.ShapeDtypeStruct((_N, _K), jnp.float32),
        ],
        scratch_shapes=[
            pltpu.VMEM((_N, _D), jnp.float32),
            pltpu.VMEM((_BR, _N), jnp.float32),
        ],
    )(mm_embedding)

    rows = jnp.broadcast_to(jnp.arange(_N)[:, None], (_N, _K)).reshape(-1)
    indices = jnp.stack((rows, knn_ind.reshape(-1)), axis=0)
    return (indices, vals.reshape(-1))


# speedup vs baseline: 5.0949x; 5.0949x over previous
import functools

import jax
import jax.numpy as jnp
from jax.experimental import pallas as pl
from jax.experimental.pallas import tpu as pltpu

_K = 10
_N = 8192
_D = 256
_BR = 256
_NBLK = _N // _BR


def _knn_kernel(x_ref, idx_ref, val_ref, xn_ref):
    i = pl.program_id(0)

    @pl.when(i == 0)
    def _normalize():
        x = x_ref[...]
        n2 = jnp.sum(x * x, axis=1, keepdims=True)
        xn_ref[...] = x / jnp.sqrt(n2)

    xn_blk = xn_ref[pl.ds(i * _BR, _BR), :]
    s = jax.lax.dot_general(
        xn_blk, xn_ref[...], (((1,), (1,)), ((), ())),
        preferred_element_type=jnp.float32)

    col_ids = jax.lax.broadcasted_iota(jnp.int32, (_BR, _N), 1)
    picked = []
    for _ in range(_K):
        idx = jnp.argmax(s, axis=1).astype(jnp.int32)[:, None]
        picked.append(idx)
        s = jnp.where(col_ids == idx, -jnp.inf, s)
    idx_ref[...] = jnp.concatenate(picked, axis=1)

    row_sum = jnp.float32(1e-7) + jnp.float32(_K)
    r_inv_sqrt = row_sum ** -0.5
    val_ref[...] = jnp.full((_BR, _K), r_inv_sqrt * r_inv_sqrt, jnp.float32)


@functools.partial(jax.jit)
def kernel(mm_embedding):
    knn_ind, vals = pl.pallas_call(
        _knn_kernel,
        grid=(_NBLK,),
        in_specs=[pl.BlockSpec((_N, _D), lambda i: (0, 0))],
        out_specs=[
            pl.BlockSpec((_BR, _K), lambda i: (i, 0)),
            pl.BlockSpec((_BR, _K), lambda i: (i, 0)),
        ],
        out_shape=[
            jax.ShapeDtypeStruct((_N, _K), jnp.int32),
            jax.ShapeDtypeStruct((_N, _K), jnp.float32),
        ],
        scratch_shapes=[pltpu.VMEM((_N, _D), jnp.float32)],
    )(mm_embedding)

    rows = jnp.broadcast_to(jnp.arange(_N)[:, None], (_N, _K)).reshape(-1)
    indices = jnp.stack((rows, knn_ind.reshape(-1)), axis=0)
    return (indices, vals.reshape(-1))
